# baseline (speedup 1.0000x reference)
import functools

import jax
import jax.numpy as jnp
from jax import lax
from jax.experimental import pallas as pl
from jax.experimental.pallas import tpu as pltpu
from jax.experimental.pallas import tpu_sc as plsc

S = 16
IB = 1024
NH = 8
NB = 12
NRAM = 1 << NB
THRESH = NH // 2
L = 16
PAIRS_PER_WAVE = 16
LOOKUPS = S * (S + 1) // 2
WAVE_WORDS = PAIRS_PER_WAVE * LOOKUPS
STREAMS_PER_WAVE = WAVE_WORDS // 128
JOFF = [j * S - (j * (j - 1)) // 2 for j in range(S)]


def _sc_body(npt, nc, tok_hbm, conn_hbm, ram_hbm, out_hbm,
             tok_v, conn_v, packed_v, aq_v, ak_v, ap_v,
             idx_v, lo_v, val_v, votes_v, out_v, sem0, sem1):
    npairs = NH * npt
    nwaves = npairs // PAIRS_PER_WAVE

    wid = lax.axis_index("s") * nc + lax.axis_index("c")

    iota = lax.iota(jnp.int32, L)
    iota_npt = iota * npt
    dvecs = [((iota - j) & (S - 1)) * npt for j in range(S)]
    mvecs = [(iota >= j).astype(jnp.int32) for j in range(S)]
    zero_v = iota * 0
    one_v = zero_v + 1

    sems = [sem0, sem1]

    pltpu.sync_copy(tok_hbm, tok_v)
    pltpu.sync_copy(conn_hbm.at[wid], conn_v)

    def _zero(nl, c):
        votes_v[pl.ds(nl * L, L)] = zero_v
        return c
    lax.fori_loop(0, npt, _zero, 0)

    def _pack(grp, c):
        acc = tok_v[0, pl.ds(grp * L, L)]
        for i in range(1, S):
            acc = acc | (tok_v[i, pl.ds(grp * L, L)] << i)
        packed_v[pl.ds(grp * L, L)] = acc
        return c
    lax.fori_loop(0, IB // L, _pack, 0)

    def _heads(h, c):
        for g in range(npt // L):
            mq, mk, pp, pm2 = [], [], [], []
            for b in range(NB):
                cv = conn_v[pl.ds((h * NB + b) * npt + g * L, L)]
                gal = plsc.load_gather(packed_v, [cv & (IB - 1)])
                qm = cv < IB
                km = (cv >= IB) & (cv < 2 * IB)
                pmask = cv >= 2 * IB
                mq.append(jnp.where(qm, gal, 0) << b)
                mk.append(jnp.where(km, gal, 0) << b)
                pp.append(cv & 3)
                pm2.append(jnp.where(pmask, 1 << b, 0))

            def _bits(i, c2):
                aqv = (mq[0] >> i) & 1
                akv = (mk[0] >> i) & 1
                for b in range(1, NB):
                    aqv = aqv + ((mq[b] >> i) & (1 << b))
                    akv = akv + ((mk[b] >> i) & (1 << b))
                ivec = zero_v + i
                apv = ((ivec >> pp[0]) & 1) * pm2[0]
                for b in range(1, NB):
                    apv = apv + (((ivec >> pp[b]) & 1) * pm2[b])
                off = (h * S + i) * npt + g * L
                aq_v[pl.ds(off, L)] = aqv
                ak_v[pl.ds(off, L)] = akv
                ap_v[pl.ds(off, L)] = apv
                return c2
            lax.fori_loop(0, S, _bits, 0)
        return c
    lax.fori_loop(0, NH, _heads, 0)

    def _build_fire(w, slot, sem):
        def _pairb(t, c1):
            pr = w * PAIRS_PER_WAVE + t
            h = pr // npt
            nl = lax.rem(pr, npt)
            hbase = h * S * npt + nl
            gbase = (h * IB + wid * npt + nl) * NRAM
            aqv = plsc.load_gather(aq_v, [iota_npt + hbase]) + gbase
            for j in range(S):
                akj = plsc.load_gather(ak_v, [zero_v + (hbase + j * npt)])
                apd = plsc.load_gather(ap_v, [dvecs[j] + hbase])
                gidx = aqv + akj + apd
                off = slot + t * LOOKUPS + JOFF[j]
                plsc.store_compressed(
                    idx_v.at[pl.ds(off, L)], gidx >> 4, mask=(iota >= j))
                plsc.store_compressed(
                    lo_v.at[pl.ds(off, L)], gidx & 15, mask=(iota >= j))
            return c1
        lax.fori_loop(0, PAIRS_PER_WAVE, _pairb, 0)

        def _fire(k, c1):
            pltpu.async_copy(
                ram_hbm.at[idx_v.at[pl.ds(slot + k * 128, 128)]],
                val_v.at[pl.ds(slot + k * 128, 128)], sem)
            return c1
        lax.fori_loop(0, STREAMS_PER_WAVE, _fire, 0)

    def _drain_reduce(w, slot, sem):
        pltpu.make_async_copy(
            ram_hbm.at[pl.ds(0, WAVE_WORDS)],
            val_v.at[pl.ds(slot, WAVE_WORDS)], sem).wait()
        def _pairr(t, c1):
            pr = w * PAIRS_PER_WAVE + t
            nl = lax.rem(pr, npt)
            parv = zero_v
            for j in range(S):
                off = slot + t * LOOKUPS + JOFF[j]
                lo = plsc.load_expanded(
                    lo_v.at[pl.ds(off, L)], mask=(iota >= j))
                rows = off + ((iota - j) & (S - 1))
                vals = plsc.load_gather(val_v, [rows, lo & (L - 1)])
                parv = parv ^ (vals & mvecs[j])
            votes_v[pl.ds(nl * L, L)] = votes_v[pl.ds(nl * L, L)] + parv
            return c1
        lax.fori_loop(0, PAIRS_PER_WAVE, _pairr, 0)

    def _waves(s_, c0):
        for par in range(2):
            w = 2 * s_ + par
            _build_fire(w, par * WAVE_WORDS, sems[par])

            @pl.when(w >= 1)
            def _():
                _drain_reduce(w - 1, (1 - par) * WAVE_WORDS, sems[1 - par])
        return c0
    lax.fori_loop(0, nwaves // 2, _waves, 0)
    _drain_reduce(nwaves - 1, ((nwaves - 1) % 2) * WAVE_WORDS,
                  sems[(nwaves - 1) % 2])

    def _out(nl, c):
        out_v[pl.ds(nl * L, L)] = jnp.where(
            votes_v[pl.ds(nl * L, L)] > THRESH, one_v, zero_v)
        return c
    lax.fori_loop(0, npt, _out, 0)
    pltpu.sync_copy(out_v, out_hbm.at[pl.ds(wid * npt * S, npt * S)])


@jax.jit
def kernel(tokens, connections, ram):
    info = plsc.get_sparse_core_info()
    nc, ns = info.num_cores, info.num_subcores
    nw = nc * ns
    npt = IB // nw

    conn_t = connections.transpose(0, 2, 1).reshape(NH, NB, nw, npt)
    conn_t = conn_t.transpose(2, 0, 1, 3).reshape(nw, NH * NB * npt)

    mesh = plsc.VectorSubcoreMesh(core_axis_name="c", subcore_axis_name="s")
    body = functools.partial(_sc_body, npt, nc)
    out_t = pl.kernel(
        body,
        out_type=jax.ShapeDtypeStruct((IB * S,), jnp.int32),
        mesh=mesh,
        compiler_params=pltpu.CompilerParams(
            needs_layout_passes=False, use_tc_tiling_on_sc=False),
        scratch_types=[
            pltpu.VMEM((S, IB), jnp.int32),
            pltpu.VMEM((NH * NB * npt,), jnp.int32),
            pltpu.VMEM((IB,), jnp.int32),
            pltpu.VMEM((NH * S * npt,), jnp.int32),
            pltpu.VMEM((NH * S * npt,), jnp.int32),
            pltpu.VMEM((NH * S * npt,), jnp.int32),
            pltpu.VMEM((2 * WAVE_WORDS + L,), jnp.int32),
            pltpu.VMEM((2 * WAVE_WORDS + L,), jnp.int32),
            pltpu.VMEM((2 * WAVE_WORDS + L, L), jnp.int32),
            pltpu.VMEM((npt * S,), jnp.int32),
            pltpu.VMEM((npt * S,), jnp.int32),
            pltpu.SemaphoreType.DMA,
            pltpu.SemaphoreType.DMA,
        ],
    )(tokens, conn_t, ram.reshape(NH * IB * NRAM // L, L))
    return out_t.reshape(IB, S).T


# speedup vs baseline: 2.0982x; 2.0982x over previous
import functools

import jax
import jax.numpy as jnp
from jax import lax
from jax.experimental import pallas as pl
from jax.experimental.pallas import tpu as pltpu
from jax.experimental.pallas import tpu_sc as plsc

S = 16
IB = 1024
NH = 8
NB = 12
NRAM = 1 << NB
THRESH = NH // 2
CHUNK_NEURONS = 8
NBUF = 3
L = 16


def _sc_body(npt, nc, tok_hbm, conn_hbm, ram_hbm, out_hbm,
             conn_v, packed_v, aq_v, ak_v, ap_v,
             ram_v0, ram_v1, ram_v2,
             votes_v, out_v, sem0, sem1, sem2):
    nsubs = npt // CHUNK_NEURONS
    nchunks = NH * nsubs

    wid = lax.axis_index("s") * nc + lax.axis_index("c")

    iota = lax.iota(jnp.int32, L)
    iota_npt = iota * npt
    dvecs = [((iota - j) & (S - 1)) * npt for j in range(S)]
    mvecs = [(iota >= j).astype(jnp.int32) for j in range(S)]
    zero_v = iota * 0
    one_v = zero_v + 1

    sems = [sem0, sem1, sem2]
    rams = [ram_v0, ram_v1, ram_v2]

    def _start(c_, pb):
        h = c_ // nsubs
        n0 = wid * npt + (c_ % nsubs) * CHUNK_NEURONS
        pltpu.async_copy(
            ram_hbm.at[h, pl.ds(n0, CHUNK_NEURONS)], rams[pb], sems[pb])

    for pb in range(NBUF - 1):
        _start(pb, pb)

    pltpu.sync_copy(tok_hbm, ram_v2.at[pl.ds(0, S * IB // NRAM)])
    pltpu.sync_copy(conn_hbm.at[wid], conn_v)

    def _zero(nl, c):
        votes_v[pl.ds(nl * L, L)] = zero_v
        return c
    lax.fori_loop(0, npt, _zero, 0)

    def _pack(grp, c):
        acc = ram_v2[0, pl.ds(grp * L, L)]
        for i in range(1, S):
            acc = acc | (ram_v2[i * IB // NRAM,
                                pl.ds((i * IB) % NRAM + grp * L, L)] << i)
        packed_v[pl.ds(grp * L, L)] = acc
        return c
    lax.fori_loop(0, IB // L, _pack, 0)

    def _heads(h, c):
        for g in range(npt // L):
            mq, mk, pp, pm2 = [], [], [], []
            for b in range(NB):
                cv = conn_v[pl.ds((h * NB + b) * npt + g * L, L)]
                gal = plsc.load_gather(packed_v, [cv & (IB - 1)])
                qm = cv < IB
                km = (cv >= IB) & (cv < 2 * IB)
                pmask = cv >= 2 * IB
                mq.append(jnp.where(qm, gal, 0) << b)
                mk.append(jnp.where(km, gal, 0) << b)
                pp.append(cv & 3)
                pm2.append(jnp.where(pmask, 1 << b, 0))

            def _bits(i, c2):
                aqv = (mq[0] >> i) & 1
                akv = (mk[0] >> i) & 1
                for b in range(1, NB):
                    aqv = aqv + ((mq[b] >> i) & (1 << b))
                    akv = akv + ((mk[b] >> i) & (1 << b))
                ivec = zero_v + i
                apv = ((ivec >> pp[0]) & 1) * pm2[0]
                for b in range(1, NB):
                    apv = apv + (((ivec >> pp[b]) & 1) * pm2[b])
                off = (h * S + i) * npt + g * L
                aq_v[pl.ds(off, L)] = aqv
                ak_v[pl.ds(off, L)] = akv
                ap_v[pl.ds(off, L)] = apv
                return c2
            lax.fori_loop(0, S, _bits, 0)
        return c
    lax.fori_loop(0, NH, _heads, 0)

    _start(NBUF - 1, NBUF - 1)

    def _sloop(s_, c0):
        for pb in range(NBUF):
            c_ = NBUF * s_ + pb

            @pl.when(c_ < nchunks)
            def _chunk():
                pltpu.make_async_copy(
                    ram_hbm.at[0, pl.ds(0, CHUNK_NEURONS)], rams[pb],
                    sems[pb]).wait()
                h = c_ // nsubs
                nl0 = (c_ % nsubs) * CHUNK_NEURONS
                hbase = h * S * npt
                for p in range(CHUNK_NEURONS):
                    nl = nl0 + p
                    aqv = plsc.load_gather(aq_v, [iota_npt + (hbase + nl)])
                    parv = zero_v
                    for j in range(S):
                        akj = plsc.load_gather(
                            ak_v, [zero_v + (hbase + j * npt + nl)])
                        apd = plsc.load_gather(
                            ap_v, [dvecs[j] + (hbase + nl)])
                        addr = aqv + akj + apd
                        val = plsc.load_gather(rams[pb], [zero_v + p, addr])
                        parv = parv ^ (val & mvecs[j])
                    votes_v[pl.ds(nl * L, L)] = (
                        votes_v[pl.ds(nl * L, L)] + parv)
                nxt = c_ + NBUF

                @pl.when(nxt < nchunks)
                def _():
                    _start(nxt, pb)
        return c0
    lax.fori_loop(0, (nchunks + NBUF - 1) // NBUF, _sloop, 0)

    def _out(nl, c):
        out_v[pl.ds(nl * L, L)] = jnp.where(
            votes_v[pl.ds(nl * L, L)] > THRESH, one_v, zero_v)
        return c
    lax.fori_loop(0, npt, _out, 0)
    pltpu.sync_copy(out_v, out_hbm.at[pl.ds(wid * npt * S, npt * S)])


@jax.jit
def kernel(tokens, connections, ram):
    info = plsc.get_sparse_core_info()
    nc, ns = info.num_cores, info.num_subcores
    nw = nc * ns
    npt = IB // nw

    conn_t = connections.transpose(0, 2, 1).reshape(NH, NB, nw, npt)
    conn_t = conn_t.transpose(2, 0, 1, 3).reshape(nw, NH * NB * npt)
    tok_f = tokens.reshape(S * IB // NRAM, NRAM)

    mesh = plsc.VectorSubcoreMesh(core_axis_name="c", subcore_axis_name="s")
    body = functools.partial(_sc_body, npt, nc)
    out_t = pl.kernel(
        body,
        out_type=jax.ShapeDtypeStruct((IB * S,), jnp.int32),
        mesh=mesh,
        compiler_params=pltpu.CompilerParams(needs_layout_passes=False),
        scratch_types=[
            pltpu.VMEM((NH * NB * npt,), jnp.int32),
            pltpu.VMEM((IB,), jnp.int32),
            pltpu.VMEM((NH * S * npt,), jnp.int32),
            pltpu.VMEM((NH * S * npt,), jnp.int32),
            pltpu.VMEM((NH * S * npt,), jnp.int32),
            pltpu.VMEM((CHUNK_NEURONS, NRAM), jnp.int32),
            pltpu.VMEM((CHUNK_NEURONS, NRAM), jnp.int32),
            pltpu.VMEM((CHUNK_NEURONS, NRAM), jnp.int32),
            pltpu.VMEM((npt * S,), jnp.int32),
            pltpu.VMEM((npt * S,), jnp.int32),
            pltpu.SemaphoreType.DMA,
            pltpu.SemaphoreType.DMA,
            pltpu.SemaphoreType.DMA,
        ],
    )(tok_f, conn_t, ram)
    return out_t.reshape(IB, S).T


# speedup vs baseline: 2.1915x; 1.0445x over previous
import functools

import jax
import jax.numpy as jnp
from jax import lax
from jax.experimental import pallas as pl
from jax.experimental.pallas import tpu as pltpu
from jax.experimental.pallas import tpu_sc as plsc

S = 16
IB = 1024
NH = 8
NB = 12
NRAM = 1 << NB
THRESH = NH // 2
CHUNK_NEURONS = 4
NBUF = 4
L = 16


def _sc_body(npt, nc, tok_hbm, conn_hbm, ram_hbm, out_hbm,
             conn_v, packed_v, aq_v, ak_v, ap_v,
             ram_v0, ram_v1, ram_v2, ram_v3,
             votes_v, out_v, sem0, sem1, sem2, sem3):
    nsubs = npt // CHUNK_NEURONS
    nchunks = NH * nsubs

    wid = lax.axis_index("s") * nc + lax.axis_index("c")

    iota = lax.iota(jnp.int32, L)
    iota_npt = iota * npt
    dvecs = [((iota - j) & (S - 1)) * npt for j in range(S)]
    mvecs = [(iota >= j).astype(jnp.int32) for j in range(S)]
    zero_v = iota * 0
    one_v = zero_v + 1

    sems = [sem0, sem1, sem2, sem3]
    rams = [ram_v0, ram_v1, ram_v2, ram_v3]

    def _start(c_, pb):
        h = c_ // nsubs
        n0 = wid * npt + (c_ % nsubs) * CHUNK_NEURONS
        pltpu.async_copy(
            ram_hbm.at[h, pl.ds(n0, CHUNK_NEURONS)], rams[pb], sems[pb])

    for pb in range(NBUF - 1):
        _start(pb, pb)

    pltpu.sync_copy(tok_hbm, ram_v3)
    pltpu.sync_copy(conn_hbm.at[wid], conn_v)

    def _zero(nl, c):
        votes_v[pl.ds(nl * L, L)] = zero_v
        return c
    lax.fori_loop(0, npt, _zero, 0)

    def _pack(grp, c):
        acc = ram_v3[0, pl.ds(grp * L, L)]
        for i in range(1, S):
            acc = acc | (ram_v3[i * IB // NRAM,
                                pl.ds((i * IB) % NRAM + grp * L, L)] << i)
        packed_v[pl.ds(grp * L, L)] = acc
        return c
    lax.fori_loop(0, IB // L, _pack, 0)

    def _heads(h, c):
        for g in range(npt // L):
            mq, mk, pp, pm2 = [], [], [], []
            for b in range(NB):
                cv = conn_v[pl.ds((h * NB + b) * npt + g * L, L)]
                gal = plsc.load_gather(packed_v, [cv & (IB - 1)])
                qm = cv < IB
                km = (cv >= IB) & (cv < 2 * IB)
                pmask = cv >= 2 * IB
                mq.append(jnp.where(qm, gal, 0) << b)
                mk.append(jnp.where(km, gal, 0) << b)
                pp.append(cv & 3)
                pm2.append(jnp.where(pmask, 1 << b, 0))

            def _bits(i, c2):
                aqv = (mq[0] >> i) & 1
                akv = (mk[0] >> i) & 1
                for b in range(1, NB):
                    aqv = aqv + ((mq[b] >> i) & (1 << b))
                    akv = akv + ((mk[b] >> i) & (1 << b))
                ivec = zero_v + i
                apv = ((ivec >> pp[0]) & 1) * pm2[0]
                for b in range(1, NB):
                    apv = apv + (((ivec >> pp[b]) & 1) * pm2[b])
                off = (h * S + i) * npt + g * L
                aq_v[pl.ds(off, L)] = aqv
                ak_v[pl.ds(off, L)] = akv
                ap_v[pl.ds(off, L)] = apv
                return c2
            lax.fori_loop(0, S, _bits, 0)
        return c
    lax.fori_loop(0, NH, _heads, 0)

    _start(NBUF - 1, NBUF - 1)

    def _sloop(s_, c0):
        for pb in range(NBUF):
            c_ = NBUF * s_ + pb
            pltpu.make_async_copy(
                ram_hbm.at[0, pl.ds(0, CHUNK_NEURONS)], rams[pb],
                sems[pb]).wait()
            h = c_ // nsubs
            nl0 = (c_ % nsubs) * CHUNK_NEURONS
            hbase = h * S * npt
            for p in range(CHUNK_NEURONS):
                nl = nl0 + p
                aqv = plsc.load_gather(aq_v, [iota_npt + (hbase + nl)])
                parv = zero_v
                for j in range(S):
                    akj = plsc.load_gather(
                        ak_v, [zero_v + (hbase + j * npt + nl)])
                    apd = plsc.load_gather(ap_v, [dvecs[j] + (hbase + nl)])
                    addr = aqv + akj + apd
                    val = plsc.load_gather(rams[pb], [zero_v + p, addr])
                    parv = parv ^ (val & mvecs[j])
                votes_v[pl.ds(nl * L, L)] = votes_v[pl.ds(nl * L, L)] + parv
            nxt = c_ + NBUF

            @pl.when(nxt < nchunks)
            def _():
                _start(nxt, pb)
        return c0
    lax.fori_loop(0, nchunks // NBUF, _sloop, 0)

    def _out(nl, c):
        out_v[pl.ds(nl * L, L)] = jnp.where(
            votes_v[pl.ds(nl * L, L)] > THRESH, one_v, zero_v)
        return c
    lax.fori_loop(0, npt, _out, 0)
    pltpu.sync_copy(out_v, out_hbm.at[pl.ds(wid * npt * S, npt * S)])


@jax.jit
def kernel(tokens, connections, ram):
    info = plsc.get_sparse_core_info()
    nc, ns = info.num_cores, info.num_subcores
    nw = nc * ns
    npt = IB // nw

    conn_t = connections.transpose(0, 2, 1).reshape(NH, NB, nw, npt)
    conn_t = conn_t.transpose(2, 0, 1, 3).reshape(nw, NH * NB * npt)
    tok_f = tokens.reshape(CHUNK_NEURONS, NRAM)

    mesh = plsc.VectorSubcoreMesh(core_axis_name="c", subcore_axis_name="s")
    body = functools.partial(_sc_body, npt, nc)
    out_t = pl.kernel(
        body,
        out_type=jax.ShapeDtypeStruct((IB * S,), jnp.int32),
        mesh=mesh,
        compiler_params=pltpu.CompilerParams(needs_layout_passes=False),
        scratch_types=[
            pltpu.VMEM((NH * NB * npt,), jnp.int32),
            pltpu.VMEM((IB,), jnp.int32),
            pltpu.VMEM((NH * S * npt,), jnp.int32),
            pltpu.VMEM((NH * S * npt,), jnp.int32),
            pltpu.VMEM((NH * S * npt,), jnp.int32),
            pltpu.VMEM((CHUNK_NEURONS, NRAM), jnp.int32),
            pltpu.VMEM((CHUNK_NEURONS, NRAM), jnp.int32),
            pltpu.VMEM((CHUNK_NEURONS, NRAM), jnp.int32),
            pltpu.VMEM((CHUNK_NEURONS, NRAM), jnp.int32),
            pltpu.VMEM((npt * S,), jnp.int32),
            pltpu.VMEM((npt * S,), jnp.int32),
            pltpu.SemaphoreType.DMA,
            pltpu.SemaphoreType.DMA,
            pltpu.SemaphoreType.DMA,
            pltpu.SemaphoreType.DMA,
        ],
    )(tok_f, conn_t, ram)
    return out_t.reshape(IB, S).T


# speedup vs baseline: 2.1985x; 1.0032x over previous
import functools

import jax
import jax.numpy as jnp
from jax import lax
from jax.experimental import pallas as pl
from jax.experimental.pallas import tpu as pltpu
from jax.experimental.pallas import tpu_sc as plsc

S = 16
IB = 1024
NH = 8
NB = 12
NRAM = 1 << NB
THRESH = NH // 2
CHUNK_NEURONS = 2
NBUF = 8
L = 16


def _sc_body(npt, nc, tok_hbm, conn_hbm, ram_hbm, out_hbm,
             tok_v, conn_v, packed_v, aq_v, ak_v, ap_v,
             ram_v0, ram_v1, ram_v2, ram_v3, ram_v4, ram_v5, ram_v6, ram_v7,
             votes_v, out_v,
             sem0, sem1, sem2, sem3, sem4, sem5, sem6, sem7):
    nsubs = npt // CHUNK_NEURONS
    nchunks = NH * nsubs

    wid = lax.axis_index("s") * nc + lax.axis_index("c")

    iota = lax.iota(jnp.int32, L)
    iota_npt = iota * npt
    dvecs = [((iota - j) & (S - 1)) * npt for j in range(S)]
    mvecs = [(iota >= j).astype(jnp.int32) for j in range(S)]
    zero_v = iota * 0
    one_v = zero_v + 1

    sems = [sem0, sem1, sem2, sem3, sem4, sem5, sem6, sem7]
    rams = [ram_v0, ram_v1, ram_v2, ram_v3, ram_v4, ram_v5, ram_v6, ram_v7]

    def _start(c_, pb):
        h = c_ // nsubs
        n0 = wid * npt + (c_ % nsubs) * CHUNK_NEURONS
        pltpu.async_copy(
            ram_hbm.at[h, pl.ds(n0, CHUNK_NEURONS)], rams[pb], sems[pb])

    for pb in range(NBUF):
        _start(pb, pb)

    pltpu.sync_copy(tok_hbm, tok_v)
    pltpu.sync_copy(conn_hbm.at[wid], conn_v)

    def _zero(nl, c):
        votes_v[pl.ds(nl * L, L)] = zero_v
        return c
    lax.fori_loop(0, npt, _zero, 0)

    def _pack(grp, c):
        acc = tok_v[0, pl.ds(grp * L, L)]
        for i in range(1, S):
            acc = acc | (tok_v[i, pl.ds(grp * L, L)] << i)
        packed_v[pl.ds(grp * L, L)] = acc
        return c
    lax.fori_loop(0, IB // L, _pack, 0)

    def _heads(h, c):
        for g in range(npt // L):
            mq, mk, pp, pm2 = [], [], [], []
            for b in range(NB):
                cv = conn_v[pl.ds((h * NB + b) * npt + g * L, L)]
                gal = plsc.load_gather(packed_v, [cv & (IB - 1)])
                qm = cv < IB
                km = (cv >= IB) & (cv < 2 * IB)
                pmask = cv >= 2 * IB
                mq.append(jnp.where(qm, gal, 0) << b)
                mk.append(jnp.where(km, gal, 0) << b)
                pp.append(cv & 3)
                pm2.append(jnp.where(pmask, 1 << b, 0))

            def _bits(i, c2):
                aqv = (mq[0] >> i) & 1
                akv = (mk[0] >> i) & 1
                for b in range(1, NB):
                    aqv = aqv + ((mq[b] >> i) & (1 << b))
                    akv = akv + ((mk[b] >> i) & (1 << b))
                ivec = zero_v + i
                apv = ((ivec >> pp[0]) & 1) * pm2[0]
                for b in range(1, NB):
                    apv = apv + (((ivec >> pp[b]) & 1) * pm2[b])
                off = (h * S + i) * npt + g * L
                aq_v[pl.ds(off, L)] = aqv
                ak_v[pl.ds(off, L)] = akv
                ap_v[pl.ds(off, L)] = apv
                return c2
            lax.fori_loop(0, S, _bits, 0)
        return c
    lax.fori_loop(0, NH, _heads, 0)

    def _sloop(s_, c0):
        for pb in range(NBUF):
            c_ = NBUF * s_ + pb
            pltpu.make_async_copy(
                ram_hbm.at[0, pl.ds(0, CHUNK_NEURONS)], rams[pb],
                sems[pb]).wait()
            h = c_ // nsubs
            nl0 = (c_ % nsubs) * CHUNK_NEURONS
            hbase = h * S * npt
            for p in range(CHUNK_NEURONS):
                nl = nl0 + p
                aqv = plsc.load_gather(aq_v, [iota_npt + (hbase + nl)])
                parv = zero_v
                for j in range(S):
                    akj = plsc.load_gather(
                        ak_v, [zero_v + (hbase + j * npt + nl)])
                    apd = plsc.load_gather(ap_v, [dvecs[j] + (hbase + nl)])
                    addr = aqv + akj + apd
                    val = plsc.load_gather(rams[pb], [zero_v + p, addr])
                    parv = parv ^ (val & mvecs[j])
                votes_v[pl.ds(nl * L, L)] = votes_v[pl.ds(nl * L, L)] + parv
            nxt = c_ + NBUF

            @pl.when(nxt < nchunks)
            def _():
                _start(nxt, pb)
        return c0
    lax.fori_loop(0, nchunks // NBUF, _sloop, 0)

    def _out(nl, c):
        out_v[pl.ds(nl * L, L)] = jnp.where(
            votes_v[pl.ds(nl * L, L)] > THRESH, one_v, zero_v)
        return c
    lax.fori_loop(0, npt, _out, 0)
    pltpu.sync_copy(out_v, out_hbm.at[pl.ds(wid * npt * S, npt * S)])


@jax.jit
def kernel(tokens, connections, ram):
    info = plsc.get_sparse_core_info()
    nc, ns = info.num_cores, info.num_subcores
    nw = nc * ns
    npt = IB // nw

    conn_t = connections.transpose(0, 2, 1).reshape(NH, NB, nw, npt)
    conn_t = conn_t.transpose(2, 0, 1, 3).reshape(nw, NH * NB * npt)

    mesh = plsc.VectorSubcoreMesh(core_axis_name="c", subcore_axis_name="s")
    body = functools.partial(_sc_body, npt, nc)
    out_t = pl.kernel(
        body,
        out_type=jax.ShapeDtypeStruct((IB * S,), jnp.int32),
        mesh=mesh,
        compiler_params=pltpu.CompilerParams(needs_layout_passes=False),
        scratch_types=[
            pltpu.VMEM((S, IB), jnp.int32),
            pltpu.VMEM((NH * NB * npt,), jnp.int32),
            pltpu.VMEM((IB,), jnp.int32),
            pltpu.VMEM((NH * S * npt,), jnp.int32),
            pltpu.VMEM((NH * S * npt,), jnp.int32),
            pltpu.VMEM((NH * S * npt,), jnp.int32),
            pltpu.VMEM((CHUNK_NEURONS, NRAM), jnp.int32),
            pltpu.VMEM((CHUNK_NEURONS, NRAM), jnp.int32),
            pltpu.VMEM((CHUNK_NEURONS, NRAM), jnp.int32),
            pltpu.VMEM((CHUNK_NEURONS, NRAM), jnp.int32),
            pltpu.VMEM((CHUNK_NEURONS, NRAM), jnp.int32),
            pltpu.VMEM((CHUNK_NEURONS, NRAM), jnp.int32),
            pltpu.VMEM((CHUNK_NEURONS, NRAM), jnp.int32),
            pltpu.VMEM((CHUNK_NEURONS, NRAM), jnp.int32),
            pltpu.VMEM((npt * S,), jnp.int32),
            pltpu.VMEM((npt * S,), jnp.int32),
            pltpu.SemaphoreType.DMA,
            pltpu.SemaphoreType.DMA,
            pltpu.SemaphoreType.DMA,
            pltpu.SemaphoreType.DMA,
            pltpu.SemaphoreType.DMA,
            pltpu.SemaphoreType.DMA,
            pltpu.SemaphoreType.DMA,
            pltpu.SemaphoreType.DMA,
        ],
    )(tokens, conn_t, ram)
    return out_t.reshape(IB, S).T
